# baseline (speedup 1.0000x reference)
import jax
import jax.numpy as jnp
from jax import lax
from jax.experimental import pallas as pl
from jax.experimental.pallas import tpu as pltpu
from jax.experimental.pallas import tpu_sc as plsc

NC = 2
NS = 16
NW = NC * NS
D = 128
DI = 64
B = 320000
EW = B // NW
C = 80
NCHUNK = EW // C


def _body(zu_hbm, zi_hbm, e_hbm, out_hbm,
          idx_u, idx_i, ru0, ri0, ru1, ri1, out_v, sem0, sem1):
    wid = lax.axis_index("s") * NC + lax.axis_index("c")

    pltpu.sync_copy(e_hbm.at[pl.ds(wid * EW, EW)], idx_u)
    pltpu.sync_copy(e_hbm.at[pl.ds(B + wid * EW, EW)], idx_i)

    def issue(c, ru, ri, sem):
        base = c * C
        pltpu.async_copy(zu_hbm.at[idx_u.at[pl.ds(base, C)]], ru, sem)
        pltpu.async_copy(zi_hbm.at[idx_i.at[pl.ds(base, C)]], ri, sem)

    def drain(ru, ri, sem):
        pltpu.make_async_copy(zu_hbm.at[pl.ds(0, C)], ru, sem).wait()
        pltpu.make_async_copy(zu_hbm.at[pl.ds(0, C)], ri, sem).wait()

    lane = lax.iota(jnp.int32, 16)
    lane4 = lane >> 2
    perms = [lane ^ k for k in (1, 2, 4, 8)]
    masks = [(lane & k) == 0 for k in (1, 2)]

    def shuffle(x, st):
        return x.at[perms[st]].get(mode="promise_in_bounds")

    def combine(x, y, st):
        xs = shuffle(x, st)
        ys = shuffle(y, st)
        m = masks[st]
        return jnp.where(m, x, ys) + jnp.where(m, xs, y)

    def compute(c, ru, ri):
        def group_body(g, _):
            row = g * 16 + lane

            def d_body(d2, carry):
                acc, _ = carry
                pacc = None
                for kk in range(4):
                    col = jnp.full((16,), d2 * 4 + kk, jnp.int32)
                    vu = plsc.load_gather(ru, [row, col])
                    vi = plsc.load_gather(ri, [row, col])
                    pb = plsc.bitcast(vu, jnp.bfloat16) * plsc.bitcast(
                        vi, jnp.bfloat16)
                    pacc = pb if pacc is None else pacc + pb
                pi = plsc.bitcast(pacc, jnp.int32)
                lo = lax.bitcast_convert_type(pi << 16, jnp.float32)
                hi = lax.bitcast_convert_type(pi, jnp.float32)
                return (acc + (lo + hi), 0)

            acc, _ = lax.fori_loop(0, DI // 4, d_body,
                                   (jnp.zeros((16,), jnp.float32), 0))
            out_v[pl.ds(c * C + g * 16, 16)] = acc
            return 0

        lax.fori_loop(0, C // 16, group_body, 0)
        for g in range(C // 16):
            x = out_v[pl.ds(c * C + g * 16, 16)]
            out_v[pl.ds(c * C + g * 16, 16)] = 1.0 / (1.0 + jnp.exp(-x))

    issue(0, ru0, ri0, sem0)
    issue(1, ru1, ri1, sem1)

    def pair_body(i, _):
        c0 = 2 * i
        drain(ru0, ri0, sem0)
        compute(c0, ru0, ri0)
        issue(c0 + 2, ru0, ri0, sem0)
        drain(ru1, ri1, sem1)
        compute(c0 + 1, ru1, ri1)
        issue(c0 + 3, ru1, ri1, sem1)
        return 0

    lax.fori_loop(0, (NCHUNK - 3) // 2, pair_body, 0)

    drain(ru0, ri0, sem0)
    compute(NCHUNK - 3, ru0, ri0)
    issue(NCHUNK - 1, ru0, ri0, sem0)
    drain(ru1, ri1, sem1)
    compute(NCHUNK - 2, ru1, ri1)
    drain(ru0, ri0, sem0)
    compute(NCHUNK - 1, ru0, ri0)

    pltpu.sync_copy(out_v, out_hbm.at[pl.ds(wid * EW, EW)])


@jax.jit
def _run(z_user, z_item, edge_index):
    k = pl.kernel(
        _body,
        mesh=plsc.VectorSubcoreMesh(core_axis_name="c", subcore_axis_name="s"),
        compiler_params=pltpu.CompilerParams(use_tc_tiling_on_sc=False, needs_layout_passes=False),
        out_type=jax.ShapeDtypeStruct((B,), jnp.float32),
        scratch_types=[
            pltpu.VMEM((EW,), jnp.int32),
            pltpu.VMEM((EW,), jnp.int32),
            pltpu.VMEM((C, DI), jnp.int32),
            pltpu.VMEM((C, DI), jnp.int32),
            pltpu.VMEM((C, DI), jnp.int32),
            pltpu.VMEM((C, DI), jnp.int32),
            pltpu.VMEM((EW,), jnp.float32),
            pltpu.SemaphoreType.DMA,
            pltpu.SemaphoreType.DMA,
        ],
    )
    return k(z_user, z_item, edge_index)


def _pack_bf16(z):
    zi = jax.lax.bitcast_convert_type(z, jnp.uint32)
    rn = (zi + jnp.uint32(0x7FFF) + ((zi >> 16) & jnp.uint32(1))) >> 16
    packed = rn[:, :DI] | (rn[:, DI:] << 16)
    return jax.lax.bitcast_convert_type(packed, jnp.int32)


def kernel(z_user, z_item, edge_index):
    return _run(_pack_bf16(z_user), _pack_bf16(z_item),
                edge_index.reshape(-1))


# speedup vs baseline: 4.2666x; 4.2666x over previous
import jax
import jax.numpy as jnp
from jax import lax
from jax.experimental import pallas as pl
from jax.experimental.pallas import tpu as pltpu
from jax.experimental.pallas import tpu_sc as plsc

NC = 2
NS = 16
NW = NC * NS
D = 128
DI = 64
B = 320000
EW = B // NW
C = 80
NCHUNK = EW // C


def _body(zu_hbm, zi_hbm, e_hbm, out_hbm,
          idx_u, idx_i, ru0, ri0, ru1, ri1, out_v, stage, sem0, sem1):
    wid = lax.axis_index("s") * NC + lax.axis_index("c")

    pltpu.sync_copy(e_hbm.at[pl.ds(wid * EW, EW)], idx_u)
    pltpu.sync_copy(e_hbm.at[pl.ds(B + wid * EW, EW)], idx_i)

    def issue(c, ru, ri, sem):
        base = c * C
        pltpu.async_copy(zu_hbm.at[idx_u.at[pl.ds(base, C)]], ru, sem)
        pltpu.async_copy(zi_hbm.at[idx_i.at[pl.ds(base, C)]], ri, sem)

    def drain(ru, ri, sem):
        pltpu.make_async_copy(zu_hbm.at[pl.ds(0, C)], ru, sem).wait()
        pltpu.make_async_copy(zu_hbm.at[pl.ds(0, C)], ri, sem).wait()

    lane = lax.iota(jnp.int32, 16)
    lane4 = lane >> 2
    perms = [lane ^ k for k in (1, 2, 4, 8)]
    masks = [(lane & k) == 0 for k in (1, 2)]

    def shuffle(x, st):
        return x.at[perms[st]].get(mode="promise_in_bounds")

    def combine(x, y, st):
        xs = shuffle(x, st)
        ys = shuffle(y, st)
        m = masks[st]
        return jnp.where(m, x, ys) + jnp.where(m, xs, y)

    reorder = (lane >> 2) * 16 + (lane & 3)

    def compute(c, ru, ri):
        def block_body(b, _):
            ss = []
            for j in range(4):
                e = b * 4 + j
                pacc = None
                for kk in range(4):
                    ub = plsc.bitcast(ru[e, pl.ds(kk * 16, 16)],
                                      jnp.bfloat16)
                    vb = plsc.bitcast(ri[e, pl.ds(kk * 16, 16)],
                                      jnp.bfloat16)
                    pb = ub * vb
                    pacc = pb if pacc is None else pacc + pb
                pi = plsc.bitcast(pacc, jnp.int32)
                lo = lax.bitcast_convert_type(pi << 16, jnp.float32)
                hi = lax.bitcast_convert_type(pi, jnp.float32)
                ss.append(lo + hi)
            t0 = combine(ss[0], ss[1], 0)
            t1 = combine(ss[2], ss[3], 0)
            t = combine(t0, t1, 1)
            t = t + shuffle(t, 2)
            t = t + shuffle(t, 3)
            stage[pl.ds(b * 16, 16)] = t
            return 0

        lax.fori_loop(0, C // 4, block_body, 0)
        for g in range(C // 16):
            v = plsc.load_gather(stage, [g * 64 + reorder])
            out_v[pl.ds(c * C + g * 16, 16)] = 1.0 / (1.0 + jnp.exp(-v))

    issue(0, ru0, ri0, sem0)
    issue(1, ru1, ri1, sem1)

    def pair_body(i, _):
        c0 = 2 * i
        drain(ru0, ri0, sem0)
        compute(c0, ru0, ri0)
        issue(c0 + 2, ru0, ri0, sem0)
        drain(ru1, ri1, sem1)
        compute(c0 + 1, ru1, ri1)
        issue(c0 + 3, ru1, ri1, sem1)
        return 0

    lax.fori_loop(0, (NCHUNK - 3) // 2, pair_body, 0)

    drain(ru0, ri0, sem0)
    compute(NCHUNK - 3, ru0, ri0)
    issue(NCHUNK - 1, ru0, ri0, sem0)
    drain(ru1, ri1, sem1)
    compute(NCHUNK - 2, ru1, ri1)
    drain(ru0, ri0, sem0)
    compute(NCHUNK - 1, ru0, ri0)

    pltpu.sync_copy(out_v, out_hbm.at[pl.ds(wid * EW, EW)])


@jax.jit
def _run(z_user, z_item, edge_index):
    k = pl.kernel(
        _body,
        mesh=plsc.VectorSubcoreMesh(core_axis_name="c", subcore_axis_name="s"),
        compiler_params=pltpu.CompilerParams(use_tc_tiling_on_sc=False, needs_layout_passes=False),
        out_type=jax.ShapeDtypeStruct((B,), jnp.float32),
        scratch_types=[
            pltpu.VMEM((EW,), jnp.int32),
            pltpu.VMEM((EW,), jnp.int32),
            pltpu.VMEM((C, DI), jnp.int32),
            pltpu.VMEM((C, DI), jnp.int32),
            pltpu.VMEM((C, DI), jnp.int32),
            pltpu.VMEM((C, DI), jnp.int32),
            pltpu.VMEM((EW,), jnp.float32),
            pltpu.VMEM((C * 4,), jnp.float32),
            pltpu.SemaphoreType.DMA,
            pltpu.SemaphoreType.DMA,
        ],
    )
    return k(z_user, z_item, edge_index)


def _pack_bf16(z):
    zi = jax.lax.bitcast_convert_type(z, jnp.uint32)
    rn = (zi + jnp.uint32(0x7FFF) + ((zi >> 16) & jnp.uint32(1))) >> 16
    packed = rn[:, :DI] | (rn[:, DI:] << 16)
    return jax.lax.bitcast_convert_type(packed, jnp.int32)


def kernel(z_user, z_item, edge_index):
    return _run(_pack_bf16(z_user), _pack_bf16(z_item),
                edge_index.reshape(-1))


# speedup vs baseline: 4.8263x; 1.1312x over previous
import jax
import jax.numpy as jnp
from jax import lax
from jax.experimental import pallas as pl
from jax.experimental.pallas import tpu as pltpu
from jax.experimental.pallas import tpu_sc as plsc

NC = 2
NS = 16
NW = NC * NS
D = 128
DI = 64
B = 320000
EW = B // NW
C = 80
NCHUNK = EW // C


def _body(zu_hbm, zi_hbm, e_hbm, out_hbm,
          idx_u, idx_i, ru0, ri0, ru1, ri1, out_v, sem0, sem1):
    wid = lax.axis_index("s") * NC + lax.axis_index("c")

    pltpu.sync_copy(e_hbm.at[pl.ds(wid * EW, EW)], idx_u)
    pltpu.sync_copy(e_hbm.at[pl.ds(B + wid * EW, EW)], idx_i)

    def issue(c, ru, ri, sem):
        base = c * C
        pltpu.async_copy(zu_hbm.at[idx_u.at[pl.ds(base, C)]], ru, sem)
        pltpu.async_copy(zi_hbm.at[idx_i.at[pl.ds(base, C)]], ri, sem)

    def drain(ru, ri, sem):
        pltpu.make_async_copy(zu_hbm.at[pl.ds(0, C)], ru, sem).wait()
        pltpu.make_async_copy(zu_hbm.at[pl.ds(0, C)], ri, sem).wait()

    lane = lax.iota(jnp.int32, 16)
    lane4 = lane >> 2
    perms = [lane ^ k for k in (1, 2, 4, 8)]
    masks = [(lane & k) == 0 for k in (1, 2)]

    def shuffle(x, st):
        return x.at[perms[st]].get(mode="promise_in_bounds")

    def combine(x, y, st):
        xs = shuffle(x, st)
        ys = shuffle(y, st)
        m = masks[st]
        return jnp.where(m, x, ys) + jnp.where(m, xs, y)

    def compute(c, ru, ri):
        def group_body(g, _):
            def block_body(b, acc):
                ss = []
                for j in range(4):
                    e = g * 16 + b * 4 + j
                    pacc = None
                    for kk in range(4):
                        ub = plsc.bitcast(ru[e, pl.ds(kk * 16, 16)],
                                          jnp.bfloat16)
                        vb = plsc.bitcast(ri[e, pl.ds(kk * 16, 16)],
                                          jnp.bfloat16)
                        pb = ub * vb
                        pacc = pb if pacc is None else pacc + pb
                    pi = plsc.bitcast(pacc, jnp.int32)
                    lo = lax.bitcast_convert_type(pi << 16, jnp.float32)
                    hi = lax.bitcast_convert_type(pi, jnp.float32)
                    ss.append(lo + hi)
                t0 = combine(ss[0], ss[1], 0)
                t1 = combine(ss[2], ss[3], 0)
                t = combine(t0, t1, 1)
                t = t + shuffle(t, 2)
                t = t + shuffle(t, 3)
                return jnp.where(lane4 == b, t, acc)

            acc = lax.fori_loop(0, 4, block_body,
                                jnp.zeros((16,), jnp.float32))
            out_v[pl.ds(c * C + g * 16, 16)] = acc
            return 0

        lax.fori_loop(0, C // 16, group_body, 0)
        for g in range(C // 16):
            x = out_v[pl.ds(c * C + g * 16, 16)]
            out_v[pl.ds(c * C + g * 16, 16)] = 1.0 / (1.0 + jnp.exp(-x))

    issue(0, ru0, ri0, sem0)
    issue(1, ru1, ri1, sem1)

    def pair_body(i, _):
        c0 = 2 * i
        drain(ru0, ri0, sem0)
        compute(c0, ru0, ri0)
        issue(c0 + 2, ru0, ri0, sem0)
        drain(ru1, ri1, sem1)
        compute(c0 + 1, ru1, ri1)
        issue(c0 + 3, ru1, ri1, sem1)
        return 0

    lax.fori_loop(0, (NCHUNK - 3) // 2, pair_body, 0)

    drain(ru0, ri0, sem0)
    compute(NCHUNK - 3, ru0, ri0)
    issue(NCHUNK - 1, ru0, ri0, sem0)
    drain(ru1, ri1, sem1)
    compute(NCHUNK - 2, ru1, ri1)
    drain(ru0, ri0, sem0)
    compute(NCHUNK - 1, ru0, ri0)

    pltpu.sync_copy(out_v, out_hbm.at[pl.ds(wid * EW, EW)])


@jax.jit
def _run(z_user, z_item, edge_index):
    k = pl.kernel(
        _body,
        mesh=plsc.VectorSubcoreMesh(core_axis_name="c", subcore_axis_name="s"),
        compiler_params=pltpu.CompilerParams(use_tc_tiling_on_sc=False, needs_layout_passes=False),
        out_type=jax.ShapeDtypeStruct((B,), jnp.float32),
        scratch_types=[
            pltpu.VMEM((EW,), jnp.int32),
            pltpu.VMEM((EW,), jnp.int32),
            pltpu.VMEM((C, DI), jnp.int32),
            pltpu.VMEM((C, DI), jnp.int32),
            pltpu.VMEM((C, DI), jnp.int32),
            pltpu.VMEM((C, DI), jnp.int32),
            pltpu.VMEM((EW,), jnp.float32),
            pltpu.SemaphoreType.DMA,
            pltpu.SemaphoreType.DMA,
        ],
    )
    return k(z_user, z_item, edge_index)


def _pack_bf16(z):
    zi = jax.lax.bitcast_convert_type(z, jnp.uint32)
    rn = (zi + jnp.uint32(0x7FFF) + ((zi >> 16) & jnp.uint32(1))) >> 16
    packed = rn[:, :DI] | (rn[:, DI:] << 16)
    return jax.lax.bitcast_convert_type(packed, jnp.int32)


def kernel(z_user, z_item, edge_index):
    return _run(_pack_bf16(z_user), _pack_bf16(z_item),
                edge_index.reshape(-1))
